# Initial kernel scaffold; baseline (speedup 1.0000x reference)
import jax
import jax.numpy as jnp
from jax.experimental import pallas as pl


def kernel(x, edge_index, W, b):
    raise NotImplementedError("write your pallas kernel here")


# speedup vs baseline: 22.1451x; 22.1451x over previous
import functools

import jax
import jax.numpy as jnp
from jax import lax
from jax.experimental import pallas as pl
from jax.experimental.pallas import tpu as pltpu, tpu_sc as plsc

N_NODES = 10000
D_FEAT = 128
N_CLASSES = 64
N_EDGES = 320000

NC, NS = 2, 16
NW = NC * NS
N_PAD = 10240
ROWS_PER_TILE = N_PAD // NS
CHUNK = 80
NCHUNK = N_EDGES // NW // CHUNK
GRID = 16
BLK = N_PAD // GRID

def _fill(ref, rows, val):
    for r in range(rows):
        for k in range(4):
            ref[r, pl.ds(k * 16, 16)] = jnp.full((16,), 1.0, jnp.float32) * val


def _deg_kernel_body(col_hbm, out_hbm, acc, idx_v, ones_v, buf_v):
    c = lax.axis_index("c")
    s = lax.axis_index("s")
    wid = c * NS + s
    base = s * ROWS_PER_TILE
    for k in range(CHUNK // 16):
        ones_v[pl.ds(k * 16, 16)] = jnp.full((16,), 1.0, jnp.float32)
    init = jnp.where(c == 0, 1.0, 0.0).astype(jnp.float32)
    for k in range(ROWS_PER_TILE // 16):
        buf_v[pl.ds(k * 16, 16)] = jnp.full((16,), 1.0, jnp.float32) * init
    pltpu.sync_copy(buf_v, acc.at[pl.ds(base, ROWS_PER_TILE)])
    plsc.subcore_barrier()
    pltpu.sync_copy(col_hbm.at[wid], idx_v)

    def body(j, carry):
        pltpu.sync_copy(ones_v, acc.at[idx_v.at[j]], add=True)
        return carry

    lax.fori_loop(0, NCHUNK, body, 0)
    plsc.subcore_barrier()
    pltpu.sync_copy(acc.at[pl.ds(base, ROWS_PER_TILE)], buf_v)
    pltpu.sync_copy(buf_v, out_hbm.at[c, pl.ds(base, ROWS_PER_TILE)])


def _hop_kernel_body(z_hbm, row_hbm, col_hbm, out_hbm, acc, idxr, idxc, gbuf, ibuf, sem):
    c = lax.axis_index("c")
    s = lax.axis_index("s")
    wid = c * NS + s
    base = s * ROWS_PER_TILE
    _fill(ibuf, CHUNK, 0.0)

    @pl.when(c == 1)
    def _():
        for k in range(ROWS_PER_TILE // CHUNK):
            pltpu.sync_copy(ibuf, acc.at[pl.ds(base + k * CHUNK, CHUNK)])

    @pl.when(c == 0)
    def _():
        for k in range(ROWS_PER_TILE // CHUNK):
            pltpu.sync_copy(z_hbm.at[pl.ds(base + k * CHUNK, CHUNK)], ibuf)
            pltpu.sync_copy(ibuf, acc.at[pl.ds(base + k * CHUNK, CHUNK)])

    plsc.subcore_barrier()
    pltpu.sync_copy(row_hbm.at[wid], idxr)
    pltpu.sync_copy(col_hbm.at[wid], idxc)

    def body(j, carry):
        pltpu.async_copy(z_hbm.at[idxr.at[j]], gbuf, sem).wait()
        pltpu.sync_copy(gbuf, acc.at[idxc.at[j]], add=True)
        return carry

    lax.fori_loop(0, NCHUNK, body, 0)
    plsc.subcore_barrier()
    for k in range(ROWS_PER_TILE // CHUNK):
        pltpu.sync_copy(acc.at[pl.ds(base + k * CHUNK, CHUNK)], gbuf)
        pltpu.sync_copy(gbuf, out_hbm.at[c, pl.ds(base + k * CHUNK, CHUNK)])


@functools.cache
def _sc_kernels():
    mesh = plsc.VectorSubcoreMesh(
        core_axis_name="c", subcore_axis_name="s", num_cores=NC, num_subcores=NS
    )
    params = pltpu.CompilerParams(use_tc_tiling_on_sc=False)
    deg = pl.kernel(
        _deg_kernel_body,
        mesh=mesh,
        compiler_params=params,
        out_type=jax.ShapeDtypeStruct((NC, N_PAD), jnp.float32),
        scratch_types=[
            pltpu.VMEM_SHARED((N_PAD,), jnp.float32),
            pltpu.VMEM((NCHUNK, CHUNK), jnp.int32),
            pltpu.VMEM((CHUNK,), jnp.float32),
            pltpu.VMEM((ROWS_PER_TILE,), jnp.float32),
        ],
    )
    hop = pl.kernel(
        _hop_kernel_body,
        mesh=mesh,
        compiler_params=params,
        out_type=jax.ShapeDtypeStruct((NC, N_PAD, N_CLASSES), jnp.float32),
        scratch_types=[
            pltpu.VMEM_SHARED((N_PAD, N_CLASSES), jnp.float32),
            pltpu.VMEM((NCHUNK, CHUNK), jnp.int32),
            pltpu.VMEM((NCHUNK, CHUNK), jnp.int32),
            pltpu.VMEM((CHUNK, N_CLASSES), jnp.float32),
            pltpu.VMEM((CHUNK, N_CLASSES), jnp.float32),
            pltpu.SemaphoreType.DMA,
        ],
    )
    return deg, hop


def _proj_body(x_ref, w_ref, da_ref, db_ref, z_ref):
    deg = da_ref[...] + db_ref[...]
    dis = lax.rsqrt(deg)
    y = lax.dot_general(
        x_ref[...], w_ref[...], (((1,), (1,)), ((), ())),
        preferred_element_type=jnp.float32,
    )
    z_ref[...] = y * dis


def _inter_body(pa_ref, pb_ref, da_ref, db_ref, t_ref):
    deg = da_ref[...] + db_ref[...]
    t_ref[...] = (pa_ref[...] + pb_ref[...]) / deg


def _final_body(pa_ref, pb_ref, da_ref, db_ref, b_ref, o_ref):
    dis = lax.rsqrt(da_ref[...] + db_ref[...])
    u = (pa_ref[...] + pb_ref[...]) * dis + b_ref[...]
    m = jnp.max(u, axis=1, keepdims=True)
    e = jnp.exp(u - m)
    lse = jnp.log(jnp.sum(e, axis=1, keepdims=True))
    o_ref[...] = u - m - lse


def _row_spec(w):
    return pl.BlockSpec((BLK, w), lambda i: (i, 0))


def _full_spec(h, w):
    return pl.BlockSpec((h, w), lambda i: (0, 0))


def kernel(x, edge_index, W, b):
    row = edge_index[0].astype(jnp.int32).reshape(NW, NCHUNK, CHUNK)
    col = edge_index[1].astype(jnp.int32).reshape(NW, NCHUNK, CHUNK)
    x_pad = jnp.pad(x, ((0, N_PAD - N_NODES), (0, 0)))
    b2 = b.reshape(1, N_CLASSES)

    _deg_kernel, _hop_kernel = _sc_kernels()
    degp = _deg_kernel(col)
    da = degp[0].reshape(N_PAD, 1)
    db = degp[1].reshape(N_PAD, 1)

    z = pl.pallas_call(
        _proj_body,
        grid=(GRID,),
        in_specs=[
            _row_spec(D_FEAT),
            _full_spec(N_CLASSES, D_FEAT),
            _row_spec(1),
            _row_spec(1),
        ],
        out_specs=_row_spec(N_CLASSES),
        out_shape=jax.ShapeDtypeStruct((N_PAD, N_CLASSES), jnp.float32),
    )(x_pad, W, da, db)

    p = _hop_kernel(z, row, col)
    t = pl.pallas_call(
        _inter_body,
        grid=(GRID,),
        in_specs=[_row_spec(N_CLASSES), _row_spec(N_CLASSES), _row_spec(1), _row_spec(1)],
        out_specs=_row_spec(N_CLASSES),
        out_shape=jax.ShapeDtypeStruct((N_PAD, N_CLASSES), jnp.float32),
    )(p[0], p[1], da, db)

    q = _hop_kernel(t, row, col)
    out = pl.pallas_call(
        _final_body,
        grid=(GRID,),
        in_specs=[
            _row_spec(N_CLASSES),
            _row_spec(N_CLASSES),
            _row_spec(1),
            _row_spec(1),
            _full_spec(1, N_CLASSES),
        ],
        out_specs=_row_spec(N_CLASSES),
        out_shape=jax.ShapeDtypeStruct((N_PAD, N_CLASSES), jnp.float32),
    )(q[0], q[1], da, db, b2)

    return out[:N_NODES]


# speedup vs baseline: 32.5522x; 1.4700x over previous
import functools

import jax
import jax.numpy as jnp
from jax import lax
from jax.experimental import pallas as pl
from jax.experimental.pallas import tpu as pltpu, tpu_sc as plsc

N_NODES = 10000
D_FEAT = 128
N_CLASSES = 64
N_EDGES = 320000

NC, NS = 2, 16
NW = NC * NS
N_PAD = 10240
ROWS_PER_TILE = N_PAD // NS
CHUNK = 80
NCHUNK = N_EDGES // NW // CHUNK
GRID = 16
BLK = N_PAD // GRID

def _fill(ref, rows, val):
    for r in range(rows):
        for k in range(4):
            ref[r, pl.ds(k * 16, 16)] = jnp.full((16,), 1.0, jnp.float32) * val


def _deg_kernel_body(col_hbm, out_hbm, acc, idx_v, ones_v, buf_v):
    c = lax.axis_index("c")
    s = lax.axis_index("s")
    wid = c * NS + s
    base = s * ROWS_PER_TILE
    for k in range(CHUNK // 16):
        ones_v[pl.ds(k * 16, 16)] = jnp.full((16,), 1.0, jnp.float32)
    init = jnp.where(c == 0, 1.0, 0.0).astype(jnp.float32)
    for k in range(ROWS_PER_TILE // 16):
        buf_v[pl.ds(k * 16, 16)] = jnp.full((16,), 1.0, jnp.float32) * init
    pltpu.sync_copy(buf_v, acc.at[pl.ds(base, ROWS_PER_TILE)])
    plsc.subcore_barrier()
    pltpu.sync_copy(col_hbm.at[wid], idx_v)

    def body(j, carry):
        pltpu.sync_copy(ones_v, acc.at[idx_v.at[j]], add=True)
        return carry

    lax.fori_loop(0, NCHUNK, body, 0)
    plsc.subcore_barrier()
    pltpu.sync_copy(
        acc.at[pl.ds(base, ROWS_PER_TILE)],
        out_hbm.at[c, pl.ds(base, ROWS_PER_TILE)],
    )


def _hop_kernel_body(z_hbm, row_hbm, col_hbm, out_hbm, acc, idxr, idxc, g0, g1, ibuf, s0, s1):
    c = lax.axis_index("c")
    s = lax.axis_index("s")
    wid = c * NS + s
    base = s * ROWS_PER_TILE
    cp_r = pltpu.async_copy(row_hbm.at[wid], idxr, s0)
    cp_c = pltpu.async_copy(col_hbm.at[wid], idxc, s1)

    @pl.when(c == 1)
    def _():
        _fill(ibuf, CHUNK, 0.0)
        for k in range(ROWS_PER_TILE // CHUNK):
            pltpu.sync_copy(ibuf, acc.at[pl.ds(base + k * CHUNK, CHUNK)])

    @pl.when(c == 0)
    def _():
        pltpu.sync_copy(
            z_hbm.at[pl.ds(base, ROWS_PER_TILE)],
            acc.at[pl.ds(base, ROWS_PER_TILE)],
        )

    cp_r.wait()
    cp_c.wait()
    plsc.subcore_barrier()

    pltpu.async_copy(z_hbm.at[idxr.at[0]], g0, s0)
    pltpu.async_copy(z_hbm.at[idxr.at[1]], g1, s1)

    def body(i, carry):
        j0 = 2 * i
        pltpu.make_async_copy(z_hbm.at[idxr.at[j0]], g0, s0).wait()
        pltpu.sync_copy(g0, acc.at[idxc.at[j0]], add=True)

        @pl.when(j0 + 2 < NCHUNK)
        def _():
            pltpu.async_copy(z_hbm.at[idxr.at[j0 + 2]], g0, s0)

        pltpu.make_async_copy(z_hbm.at[idxr.at[j0 + 1]], g1, s1).wait()
        pltpu.sync_copy(g1, acc.at[idxc.at[j0 + 1]], add=True)

        @pl.when(j0 + 3 < NCHUNK)
        def _():
            pltpu.async_copy(z_hbm.at[idxr.at[j0 + 3]], g1, s1)

        return carry

    lax.fori_loop(0, NCHUNK // 2, body, 0)
    jlast = NCHUNK - 1
    pltpu.make_async_copy(z_hbm.at[idxr.at[jlast]], g0, s0).wait()
    pltpu.sync_copy(g0, acc.at[idxc.at[jlast]], add=True)

    plsc.subcore_barrier()
    pltpu.sync_copy(
        acc.at[pl.ds(base, ROWS_PER_TILE)],
        out_hbm.at[c, pl.ds(base, ROWS_PER_TILE)],
    )


@functools.cache
def _sc_kernels():
    mesh = plsc.VectorSubcoreMesh(
        core_axis_name="c", subcore_axis_name="s", num_cores=NC, num_subcores=NS
    )
    params = pltpu.CompilerParams(use_tc_tiling_on_sc=False)
    deg = pl.kernel(
        _deg_kernel_body,
        mesh=mesh,
        compiler_params=params,
        out_type=jax.ShapeDtypeStruct((NC, N_PAD), jnp.float32),
        scratch_types=[
            pltpu.VMEM_SHARED((N_PAD,), jnp.float32),
            pltpu.VMEM((NCHUNK, CHUNK), jnp.int32),
            pltpu.VMEM((CHUNK,), jnp.float32),
            pltpu.VMEM((ROWS_PER_TILE,), jnp.float32),
        ],
    )
    hop = pl.kernel(
        _hop_kernel_body,
        mesh=mesh,
        compiler_params=params,
        out_type=jax.ShapeDtypeStruct((NC, N_PAD, N_CLASSES), jnp.float32),
        scratch_types=[
            pltpu.VMEM_SHARED((N_PAD, N_CLASSES), jnp.float32),
            pltpu.VMEM((NCHUNK, CHUNK), jnp.int32),
            pltpu.VMEM((NCHUNK, CHUNK), jnp.int32),
            pltpu.VMEM((CHUNK, N_CLASSES), jnp.float32),
            pltpu.VMEM((CHUNK, N_CLASSES), jnp.float32),
            pltpu.VMEM((CHUNK, N_CLASSES), jnp.float32),
            pltpu.SemaphoreType.DMA,
            pltpu.SemaphoreType.DMA,
        ],
    )
    return deg, hop


def _proj_body(x_ref, w_ref, da_ref, db_ref, z_ref):
    deg = da_ref[...] + db_ref[...]
    dis = lax.rsqrt(deg)
    y = lax.dot_general(
        x_ref[...], w_ref[...], (((1,), (1,)), ((), ())),
        preferred_element_type=jnp.float32,
    )
    z_ref[...] = y * dis


def _inter_body(pa_ref, pb_ref, da_ref, db_ref, t_ref):
    deg = da_ref[...] + db_ref[...]
    t_ref[...] = (pa_ref[...] + pb_ref[...]) / deg


def _final_body(pa_ref, pb_ref, da_ref, db_ref, b_ref, o_ref):
    dis = lax.rsqrt(da_ref[...] + db_ref[...])
    u = (pa_ref[...] + pb_ref[...]) * dis + b_ref[...]
    m = jnp.max(u, axis=1, keepdims=True)
    e = jnp.exp(u - m)
    lse = jnp.log(jnp.sum(e, axis=1, keepdims=True))
    o_ref[...] = u - m - lse


def _row_spec(w):
    return pl.BlockSpec((BLK, w), lambda i: (i, 0))


def _full_spec(h, w):
    return pl.BlockSpec((h, w), lambda i: (0, 0))


def kernel(x, edge_index, W, b):
    row = edge_index[0].astype(jnp.int32).reshape(NW, NCHUNK, CHUNK)
    col = edge_index[1].astype(jnp.int32).reshape(NW, NCHUNK, CHUNK)
    x_pad = jnp.pad(x, ((0, N_PAD - N_NODES), (0, 0)))
    b2 = b.reshape(1, N_CLASSES)

    _deg_kernel, _hop_kernel = _sc_kernels()
    degp = _deg_kernel(col)
    da = degp[0].reshape(N_PAD, 1)
    db = degp[1].reshape(N_PAD, 1)

    z = pl.pallas_call(
        _proj_body,
        grid=(GRID,),
        in_specs=[
            _row_spec(D_FEAT),
            _full_spec(N_CLASSES, D_FEAT),
            _row_spec(1),
            _row_spec(1),
        ],
        out_specs=_row_spec(N_CLASSES),
        out_shape=jax.ShapeDtypeStruct((N_PAD, N_CLASSES), jnp.float32),
    )(x_pad, W, da, db)

    p = _hop_kernel(z, row, col)
    t = pl.pallas_call(
        _inter_body,
        grid=(GRID,),
        in_specs=[_row_spec(N_CLASSES), _row_spec(N_CLASSES), _row_spec(1), _row_spec(1)],
        out_specs=_row_spec(N_CLASSES),
        out_shape=jax.ShapeDtypeStruct((N_PAD, N_CLASSES), jnp.float32),
    )(p[0], p[1], da, db)

    q = _hop_kernel(t, row, col)
    out = pl.pallas_call(
        _final_body,
        grid=(GRID,),
        in_specs=[
            _row_spec(N_CLASSES),
            _row_spec(N_CLASSES),
            _row_spec(1),
            _row_spec(1),
            _full_spec(1, N_CLASSES),
        ],
        out_specs=_row_spec(N_CLASSES),
        out_shape=jax.ShapeDtypeStruct((N_PAD, N_CLASSES), jnp.float32),
    )(q[0], q[1], da, db, b2)

    return out[:N_NODES]
